# baseline (speedup 1.0000x reference)
import functools

import jax
import jax.numpy as jnp
import numpy as np
from jax import lax
from jax.experimental import pallas as pl
from jax.experimental.pallas import tpu as pltpu
from jax.experimental.pallas import tpu_sc as plsc

_B, _S, _D = 4, 8192, 1024
_K = _S // 2
_BS = 2048
_NSTEPS = (_B * _S) // _BS
_ROWS = _B * _S
_IMIN = np.int32(-2147483648)

_NC, _NSUB = 2, 16
_NW = _NC * _NSUB
_TPB = _NW // _B
_CH = _S // _TPB
_NSL = _S // 16


def _tc_body(x_ref, w_ref, logits_ref, weights_ref, aux_ref, acc_ref):
    i = pl.program_id(0)
    xb = x_ref[...]
    wv = w_ref[...]
    lg = jax.lax.dot_general(
        wv, xb, (((1,), (1,)), ((), ())),
        preferred_element_type=jnp.float32)

    wgt = 1.0 / (1.0 + jnp.exp(-lg))
    weights_ref[...] = wgt[None]

    @pl.when(i == 0)
    def _init():
        acc_ref[0] = 0.0
        acc_ref[1] = 0.0

    v = wgt - 0.5
    acc_ref[0] += jnp.sum(v)
    acc_ref[1] += jnp.sum(v * v)

    b = i // (_S // _BS)
    c = i % (_S // _BS)
    logits_ref[pl.ds(b, 1), pl.ds(c * _BS, _BS)] = lg

    @pl.when(i == _NSTEPS - 1)
    def _fin():
        n = jnp.float32(_ROWS)
        aux_ref[0, 0] = (acc_ref[1] - acc_ref[0] * acc_ref[0] / n) / (n - 1.0)


def _skey(v):
    ib = lax.bitcast_convert_type(v + jnp.float32(0.0), jnp.int32)
    return jnp.where(ib < 0, ib ^ jnp.int32(0x7FFFFFFF), ib)


def _sc_body(logits_hbm, mask_hbm, row_v, ckey_v, cidx_v, hist_v, csum_v,
             mask_v):
    wid = lax.axis_index("s") * _NC + lax.axis_index("c")
    b = wid // _TPB
    ch = wid % _TPB
    iota16 = lax.iota(jnp.int32, 16)
    zeros16 = jnp.zeros((16,), jnp.int32)
    ones16 = jnp.ones((16,), jnp.int32)

    pltpu.sync_copy(logits_hbm.at[pl.ds(b * _S, _S)], row_v)

    def z_body(i, c):
        hist_v[pl.ds(i * 16, 16)] = zeros16
        return c
    lax.fori_loop(0, 16, z_body, 0)

    def h_body(j, c):
        uk = _skey(row_v[pl.ds(j * 16, 16)]) ^ _IMIN
        bins = lax.shift_right_logical(uk, 24)
        plsc.addupdate_scatter(hist_v, [bins], ones16)
        return c
    lax.fori_loop(0, _NSL, h_body, 0)

    def c_body(i, st):
        carry, condcnt = st
        h = hist_v[pl.ds(i * 16, 16)]
        incl = plsc.cumsum(h) + carry
        csum_v[pl.ds(i * 16, 16)] = incl
        cond = ((jnp.int32(_S) - incl) + h >= jnp.int32(_K))
        return (jnp.max(incl), condcnt + jnp.sum(cond.astype(jnp.int32)))
    _, condcnt = lax.fori_loop(0, 16, c_body,
                               (jnp.int32(0), jnp.int32(0)))
    f = condcnt - 1
    cf = jnp.max(plsc.load_gather(csum_v, [jnp.full((16,), f, jnp.int32)]))
    big_g = jnp.int32(_S) - cf
    kp = jnp.int32(_K) - big_g

    def comp_body(j, nc):
        uk = _skey(row_v[pl.ds(j * 16, 16)]) ^ _IMIN
        m = lax.shift_right_logical(uk, 24) == f
        plsc.store_compressed(ckey_v.at[pl.ds(nc, 16)], uk, mask=m)
        plsc.store_compressed(cidx_v.at[pl.ds(nc, 16)], iota16 + j * 16,
                              mask=m)
        return nc + jnp.sum(m.astype(jnp.int32))
    ncand = lax.fori_loop(0, _NSL, comp_body, jnp.int32(0))
    nsl = (ncand + 15) // 16
    base = lax.shift_left(f, 24)

    def bit_body(t, pfx):
        cand = pfx | lax.shift_left(jnp.int32(1), 23 - t)
        scand = (base | cand) ^ _IMIN
        def cnt_body(j, acc):
            sk = ckey_v[pl.ds(j * 16, 16)] ^ _IMIN
            valid = (iota16 + j * 16) < ncand
            return acc + ((sk >= scand) & valid).astype(jnp.int32)
        cnt = jnp.sum(lax.fori_loop(0, nsl, cnt_body, zeros16))
        return jnp.where(cnt >= kp, cand, pfx)
    pfx = lax.fori_loop(0, 24, bit_body, jnp.int32(0))
    s_t = (base | pfx) ^ _IMIN

    def gt_body(j, acc):
        sk = ckey_v[pl.ds(j * 16, 16)] ^ _IMIN
        valid = (iota16 + j * 16) < ncand
        return acc + ((sk > s_t) & valid).astype(jnp.int32)
    ngt = jnp.sum(lax.fori_loop(0, nsl, gt_body, zeros16))
    need = kp - ngt

    def eq_body(j, st):
        neq, best = st
        sk = ckey_v[pl.ds(j * 16, 16)] ^ _IMIN
        valid = (iota16 + j * 16) < ncand
        eq = (sk == s_t) & valid
        cum = plsc.cumsum(eq.astype(jnp.int32)) + neq
        gi = cidx_v[pl.ds(j * 16, 16)]
        hit = eq & (cum == need)
        best = jnp.minimum(best, jnp.min(jnp.where(hit, gi,
                                                   jnp.int32(0x40000000))))
        return (neq + jnp.sum(eq.astype(jnp.int32)), best)
    _, ineed = lax.fori_loop(0, nsl, eq_body,
                             (jnp.int32(0), jnp.int32(0x40000000)))
    jstar = ineed + 1

    coff = ch * _CH
    def m_body(j, c):
        sk = _skey(row_v[pl.ds(coff + j * 16, 16)])
        gi = iota16 + (coff + j * 16)
        sel = (sk > s_t) | ((sk == s_t) & (gi < jstar))
        mask_v[pl.ds(j * 16, 16)] = jnp.where(sel, jnp.float32(1.0),
                                              jnp.float32(0.0))
        return c
    lax.fori_loop(0, _CH // 16, m_body, 0)
    pltpu.sync_copy(mask_v, mask_hbm.at[pl.ds(b * _S + coff, _CH)])


@functools.partial(
    pl.kernel,
    out_type=jax.ShapeDtypeStruct((_ROWS,), jnp.float32),
    mesh=plsc.VectorSubcoreMesh(core_axis_name="c", subcore_axis_name="s",
                                num_cores=_NC, num_subcores=_NSUB),
    compiler_params=pltpu.CompilerParams(needs_layout_passes=False),
    scratch_types=[
        pltpu.VMEM((_S,), jnp.float32),
        pltpu.VMEM((_S + 16,), jnp.int32),
        pltpu.VMEM((_S + 16,), jnp.int32),
        pltpu.VMEM((256,), jnp.int32),
        pltpu.VMEM((256,), jnp.int32),
        pltpu.VMEM((_CH,), jnp.float32),
    ],
)
def _sc_mask(logits_hbm, mask_hbm, row_v, ckey_v, cidx_v, hist_v, csum_v,
             mask_v):
    _sc_body(logits_hbm, mask_hbm, row_v, ckey_v, cidx_v, hist_v, csum_v,
             mask_v)


def kernel(x, W):
    xf = x.reshape(_ROWS, _D)
    logits2d, w3d, aux = pl.pallas_call(
        _tc_body,
        grid=(_NSTEPS,),
        in_specs=[
            pl.BlockSpec((_BS, _D), lambda i: (i, 0)),
            pl.BlockSpec((1, _D), lambda i: (0, 0)),
        ],
        out_specs=[
            pl.BlockSpec((_B, _S), lambda i: (0, 0)),
            pl.BlockSpec((1, 1, _BS), lambda i: (i, 0, 0)),
            pl.BlockSpec(memory_space=pltpu.SMEM),
        ],
        out_shape=[
            jax.ShapeDtypeStruct((_B, _S), jnp.float32),
            jax.ShapeDtypeStruct((_NSTEPS, 1, _BS), jnp.float32),
            jax.ShapeDtypeStruct((1, 1), jnp.float32),
        ],
        scratch_shapes=[
            pltpu.SMEM((2,), jnp.float32),
        ],
        compiler_params=pltpu.CompilerParams(
            dimension_semantics=("arbitrary",)),
    )(xf, W)
    mask = _sc_mask(logits2d.reshape(_ROWS)).reshape(_B, _S, 1)
    weights = w3d.reshape(_B, _S, 1)
    aux_loss = aux.reshape(())
    return (mask, weights, aux_loss)


# speedup vs baseline: 1.1343x; 1.1343x over previous
import functools

import jax
import jax.numpy as jnp
import numpy as np
from jax import lax
from jax.experimental import pallas as pl
from jax.experimental.pallas import tpu as pltpu
from jax.experimental.pallas import tpu_sc as plsc

_B, _S, _D = 4, 8192, 1024
_K = _S // 2
_BS = 2048
_NSTEPS = (_B * _S) // _BS
_ROWS = _B * _S
_IMIN = np.int32(-2147483648)

_NC, _NSUB = 2, 16
_NW = _NC * _NSUB
_TPB = _NW // _B
_CH = _S // _TPB
_NSL = _S // 16


def _tc_body(x_ref, w_ref, logits_ref, weights_ref, aux_ref, acc_ref):
    i = pl.program_id(0)
    xb = x_ref[...]
    wv = w_ref[...]
    lg = jax.lax.dot_general(
        wv, xb, (((1,), (1,)), ((), ())),
        preferred_element_type=jnp.float32)

    wgt = 1.0 / (1.0 + jnp.exp(-lg))
    weights_ref[...] = wgt[None]

    @pl.when(i == 0)
    def _init():
        acc_ref[0] = 0.0
        acc_ref[1] = 0.0

    v = wgt - 0.5
    acc_ref[0] += jnp.sum(v)
    acc_ref[1] += jnp.sum(v * v)

    b = i // (_S // _BS)
    c = i % (_S // _BS)
    logits_ref[pl.ds(b, 1), pl.ds(c * _BS, _BS)] = lg

    @pl.when(i == _NSTEPS - 1)
    def _fin():
        n = jnp.float32(_ROWS)
        aux_ref[0, 0] = (acc_ref[1] - acc_ref[0] * acc_ref[0] / n) / (n - 1.0)


def _skey(v):
    ib = lax.bitcast_convert_type(v + jnp.float32(0.0), jnp.int32)
    return jnp.where(ib < 0, ib ^ jnp.int32(0x7FFFFFFF), ib)


def _sc_body(logits_hbm, mask_hbm, row_v, ckey_v, cidx_v, hist_v, csum_v,
             mask_v):
    wid = lax.axis_index("s") * _NC + lax.axis_index("c")
    b = wid // _TPB
    ch = wid % _TPB
    iota16 = lax.iota(jnp.int32, 16)
    zeros16 = jnp.zeros((16,), jnp.int32)
    ones16 = jnp.ones((16,), jnp.int32)

    pltpu.sync_copy(logits_hbm.at[pl.ds(b * _S, _S)], row_v)

    def z_body(i, c):
        hist_v[pl.ds(i * 16, 16)] = zeros16
        return c
    lax.fori_loop(0, 16, z_body, 0)

    @plsc.parallel_loop(0, _NSL, step=1, unroll=8)
    def _hist_loop(j):
        uk = _skey(row_v[pl.ds(j * 16, 16)]) ^ _IMIN
        bins = lax.shift_right_logical(uk, 24)
        plsc.addupdate_scatter(hist_v, [bins], ones16)

    def c_body(i, st):
        carry, condcnt = st
        h = hist_v[pl.ds(i * 16, 16)]
        incl = plsc.cumsum(h) + carry
        csum_v[pl.ds(i * 16, 16)] = incl
        cond = ((jnp.int32(_S) - incl) + h >= jnp.int32(_K))
        return (jnp.max(incl), condcnt + jnp.sum(cond.astype(jnp.int32)))
    _, condcnt = lax.fori_loop(0, 16, c_body,
                               (jnp.int32(0), jnp.int32(0)))
    f = condcnt - 1
    cf = jnp.max(plsc.load_gather(csum_v, [jnp.full((16,), f, jnp.int32)]))
    big_g = jnp.int32(_S) - cf
    kp = jnp.int32(_K) - big_g

    @plsc.parallel_loop(0, _NSL, step=1, unroll=4, carry=jnp.int32(0))
    def ncand(j, nc):
        uk = _skey(row_v[pl.ds(j * 16, 16)]) ^ _IMIN
        m = lax.shift_right_logical(uk, 24) == f
        plsc.store_compressed(ckey_v.at[pl.ds(nc, 16)], uk, mask=m)
        plsc.store_compressed(cidx_v.at[pl.ds(nc, 16)], iota16 + j * 16,
                              mask=m)
        return nc + jnp.sum(m.astype(jnp.int32))
    nsl = (ncand + 15) // 16
    base = lax.shift_left(f, 24)

    def bit_body(t, pfx):
        cand = pfx | lax.shift_left(jnp.int32(1), 23 - t)
        scand = (base | cand) ^ _IMIN
        def cnt_body(j, acc):
            sk = ckey_v[pl.ds(j * 16, 16)] ^ _IMIN
            valid = (iota16 + j * 16) < ncand
            return acc + ((sk >= scand) & valid).astype(jnp.int32)
        cnt = jnp.sum(lax.fori_loop(0, nsl, cnt_body, zeros16))
        return jnp.where(cnt >= kp, cand, pfx)
    pfx = lax.fori_loop(0, 24, bit_body, jnp.int32(0))
    s_t = (base | pfx) ^ _IMIN

    def gt_body(j, acc):
        sk = ckey_v[pl.ds(j * 16, 16)] ^ _IMIN
        valid = (iota16 + j * 16) < ncand
        return acc + ((sk > s_t) & valid).astype(jnp.int32)
    ngt = jnp.sum(lax.fori_loop(0, nsl, gt_body, zeros16))
    need = kp - ngt

    def eq_body(j, st):
        neq, best = st
        sk = ckey_v[pl.ds(j * 16, 16)] ^ _IMIN
        valid = (iota16 + j * 16) < ncand
        eq = (sk == s_t) & valid
        cum = plsc.cumsum(eq.astype(jnp.int32)) + neq
        gi = cidx_v[pl.ds(j * 16, 16)]
        hit = eq & (cum == need)
        best = jnp.minimum(best, jnp.min(jnp.where(hit, gi,
                                                   jnp.int32(0x40000000))))
        return (neq + jnp.sum(eq.astype(jnp.int32)), best)
    _, ineed = lax.fori_loop(0, nsl, eq_body,
                             (jnp.int32(0), jnp.int32(0x40000000)))
    jstar = ineed + 1

    coff = ch * _CH
    @plsc.parallel_loop(0, _CH // 16, step=1, unroll=4)
    def _mask_loop(j):
        sk = _skey(row_v[pl.ds(coff + j * 16, 16)])
        gi = iota16 + (coff + j * 16)
        sel = (sk > s_t) | ((sk == s_t) & (gi < jstar))
        mask_v[pl.ds(j * 16, 16)] = jnp.where(sel, jnp.float32(1.0),
                                              jnp.float32(0.0))
    pltpu.sync_copy(mask_v, mask_hbm.at[pl.ds(b * _S + coff, _CH)])


@functools.partial(
    pl.kernel,
    out_type=jax.ShapeDtypeStruct((_ROWS,), jnp.float32),
    mesh=plsc.VectorSubcoreMesh(core_axis_name="c", subcore_axis_name="s",
                                num_cores=_NC, num_subcores=_NSUB),
    compiler_params=pltpu.CompilerParams(needs_layout_passes=False),
    scratch_types=[
        pltpu.VMEM((_S,), jnp.float32),
        pltpu.VMEM((_S + 16,), jnp.int32),
        pltpu.VMEM((_S + 16,), jnp.int32),
        pltpu.VMEM((256,), jnp.int32),
        pltpu.VMEM((256,), jnp.int32),
        pltpu.VMEM((_CH,), jnp.float32),
    ],
)
def _sc_mask(logits_hbm, mask_hbm, row_v, ckey_v, cidx_v, hist_v, csum_v,
             mask_v):
    _sc_body(logits_hbm, mask_hbm, row_v, ckey_v, cidx_v, hist_v, csum_v,
             mask_v)


def kernel(x, W):
    xf = x.reshape(_ROWS, _D)
    logits2d, w3d, aux = pl.pallas_call(
        _tc_body,
        grid=(_NSTEPS,),
        in_specs=[
            pl.BlockSpec((_BS, _D), lambda i: (i, 0)),
            pl.BlockSpec((1, _D), lambda i: (0, 0)),
        ],
        out_specs=[
            pl.BlockSpec((_B, _S), lambda i: (0, 0)),
            pl.BlockSpec((1, 1, _BS), lambda i: (i, 0, 0)),
            pl.BlockSpec(memory_space=pltpu.SMEM),
        ],
        out_shape=[
            jax.ShapeDtypeStruct((_B, _S), jnp.float32),
            jax.ShapeDtypeStruct((_NSTEPS, 1, _BS), jnp.float32),
            jax.ShapeDtypeStruct((1, 1), jnp.float32),
        ],
        scratch_shapes=[
            pltpu.SMEM((2,), jnp.float32),
        ],
        compiler_params=pltpu.CompilerParams(
            dimension_semantics=("arbitrary",)),
    )(xf, W)
    mask = _sc_mask(logits2d.reshape(_ROWS)).reshape(_B, _S, 1)
    weights = w3d.reshape(_B, _S, 1)
    aux_loss = aux.reshape(())
    return (mask, weights, aux_loss)
